# Initial kernel scaffold; baseline (speedup 1.0000x reference)
import jax
import jax.numpy as jnp
from jax.experimental import pallas as pl


def kernel(node_features, edge_index, node_positions, node_radii, node_types, Wc1, bc1, Wc2, bc2, Wr1, br1, Wr2, br2, Wr3, br3):
    raise NotImplementedError("write your pallas kernel here")


# speedup vs baseline: 2.7492x; 2.7492x over previous
import functools

import jax
import jax.numpy as jnp
from jax import lax
from jax.experimental import pallas as pl
from jax.experimental.pallas import tpu as pltpu
from jax.experimental.pallas import tpu_sc as plsc

_N = 10000
_E = 320000
_D = 128
_NP = 10240
_NT = 16
_EPT = _E // _NT
_ECH = 4000
_NPT = _NP // _NT
_GCH = 128

_BN = 1024
_NB = _NP // _BN

_SC_MESH = plsc.VectorSubcoreMesh(
    core_axis_name="c", subcore_axis_name="s", num_cores=1, num_subcores=_NT)


def _sc_body(src_hbm, dst_hbm, px_hbm, py_hbm, pz_hbm, rr_hbm, ty_hbm,
             feat_hbm,
             bif_hbm, dot_hbm, n12_hbm, mv_hbm, xg1_hbm, xg2_hbm,
             c1_loc, c2_loc, srcb, dstb, pxt, pyt, pzt, rrt,
             tyb, pb1, pb2, mc1, mc2, bifb, dotb, n12b, mvb, rows,
             sh1, sh2, sem):
  tid = lax.axis_index("s")
  neg1 = jnp.full((16,), -1, jnp.int32)

  def init_loop(i, _):
    c1_loc[pl.ds(i * 16, 16)] = neg1
    c2_loc[pl.ds(i * 16, 16)] = neg1
    return 0

  lax.fori_loop(0, _NP // 16, init_loop, 0)

  ebase = tid * _EPT

  def chunk_loop(ci, _):
    off = ebase + ci * _ECH
    pltpu.sync_copy(src_hbm.at[pl.ds(off, _ECH)], srcb)
    pltpu.sync_copy(dst_hbm.at[pl.ds(off, _ECH)], dstb)

    def vec_loop(vi, _):
      sl = pl.ds(vi * 16, 16)
      s = srcb[sl]
      d = dstb[sl]
      occ, _last = plsc.scan_count(s)
      cur1 = plsc.load_gather(c1_loc, [s])
      cur2 = plsc.load_gather(c2_loc, [s])
      e1 = cur1 < 0
      e2 = cur2 < 0
      m1 = e1 & (occ == 1)
      m2 = ((~e1) & e2 & (occ == 1)) | (e1 & (occ == 2))
      plsc.store_scatter(c1_loc, [s], d, mask=m1)
      plsc.store_scatter(c2_loc, [s], d, mask=m2)
      return 0

    lax.fori_loop(0, _ECH // 16, vec_loop, 0)
    return 0

  lax.fori_loop(0, _EPT // _ECH, chunk_loop, 0)

  pltpu.sync_copy(c1_loc, sh1.at[tid])
  pltpu.sync_copy(c2_loc, sh2.at[tid])
  plsc.subcore_barrier()

  nb = tid * _NPT

  def minit(i, _):
    mc1[pl.ds(i * 16, 16)] = neg1
    mc2[pl.ds(i * 16, 16)] = neg1
    return 0

  lax.fori_loop(0, _NPT // 16, minit, 0)

  def tmerge(tp, _):
    pltpu.sync_copy(sh1.at[tp, pl.ds(nb, _NPT)], pb1)
    pltpu.sync_copy(sh2.at[tp, pl.ds(nb, _NPT)], pb2)

    def mvec(vi, _):
      sl = pl.ds(vi * 16, 16)
      a = pb1[sl]
      b = pb2[sl]
      c1v = mc1[sl]
      c2v = mc2[sl]
      mc1[sl] = jnp.where(c1v < 0, a, c1v)
      mc2[sl] = jnp.where(c1v < 0, b, jnp.where(c2v < 0, a, c2v))
      return 0

    lax.fori_loop(0, _NPT // 16, mvec, 0)
    return 0

  lax.fori_loop(0, _NT, tmerge, 0)

  pltpu.sync_copy(px_hbm, pxt)
  pltpu.sync_copy(py_hbm, pyt)
  pltpu.sync_copy(pz_hbm, pzt)
  pltpu.sync_copy(rr_hbm, rrt)
  pltpu.sync_copy(ty_hbm.at[pl.ds(nb, _NPT)], tyb)

  def fvec(vi, _):
    sl = pl.ds(vi * 16, 16)
    osl = pl.ds(nb + vi * 16, 16)
    c1v = mc1[sl]
    c2v = mc2[sl]
    tyv = tyb[sl]
    c1f = jnp.maximum(c1v, 0)
    c2f = jnp.where(c2v < 0, c1f, c2v)
    mc1[sl] = c1f
    mc2[sl] = c2f
    bifb[sl] = jnp.where((tyv == 1) & (c2v >= 0), 1.0, 0.0)
    p1x = plsc.load_gather(pxt, [c1f])
    p2x = plsc.load_gather(pxt, [c2f])
    p1y = plsc.load_gather(pyt, [c1f])
    p2y = plsc.load_gather(pyt, [c2f])
    p1z = plsc.load_gather(pzt, [c1f])
    p2z = plsc.load_gather(pzt, [c2f])
    ox = pxt[osl]
    oy = pyt[osl]
    oz = pzt[osl]
    v1x = p1x - ox
    v1y = p1y - oy
    v1z = p1z - oz
    v2x = p2x - ox
    v2y = p2y - oy
    v2z = p2z - oz
    dotb[sl] = v1x * v2x + v1y * v2y + v1z * v2z
    n12b[sl] = ((v1x * v1x + v1y * v1y + v1z * v1z) *
                (v2x * v2x + v2y * v2y + v2z * v2z))
    r1 = plsc.load_gather(rrt, [c1f])
    r2 = plsc.load_gather(rrt, [c2f])
    ro = rrt[osl]
    act = ro * ro * ro
    mvb[sl] = jnp.abs(r1 * r1 * r1 + r2 * r2 * r2 - act) / (act + 1e-12)
    return 0

  lax.fori_loop(0, _NPT // 16, fvec, 0)

  pltpu.sync_copy(bifb, bif_hbm.at[pl.ds(nb, _NPT)])
  pltpu.sync_copy(dotb, dot_hbm.at[pl.ds(nb, _NPT)])
  pltpu.sync_copy(n12b, n12_hbm.at[pl.ds(nb, _NPT)])
  pltpu.sync_copy(mvb, mv_hbm.at[pl.ds(nb, _NPT)])

  for ch in range(_NPT // _GCH):
    pltpu.async_copy(
        feat_hbm.at[mc1.at[pl.ds(ch * _GCH, _GCH)]], rows, sem).wait()
    pltpu.sync_copy(rows, xg1_hbm.at[pl.ds(nb + ch * _GCH, _GCH)])
    pltpu.async_copy(
        feat_hbm.at[mc2.at[pl.ds(ch * _GCH, _GCH)]], rows, sem).wait()
    pltpu.sync_copy(rows, xg2_hbm.at[pl.ds(nb + ch * _GCH, _GCH)])


_sc_call = pl.kernel(
    _sc_body,
    out_type=[
        jax.ShapeDtypeStruct((_NP,), jnp.float32),
        jax.ShapeDtypeStruct((_NP,), jnp.float32),
        jax.ShapeDtypeStruct((_NP,), jnp.float32),
        jax.ShapeDtypeStruct((_NP,), jnp.float32),
        jax.ShapeDtypeStruct((_NP, _D), jnp.float32),
        jax.ShapeDtypeStruct((_NP, _D), jnp.float32),
    ],
    mesh=_SC_MESH,
    scratch_types=[
        pltpu.VMEM((_NP,), jnp.int32),
        pltpu.VMEM((_NP,), jnp.int32),
        pltpu.VMEM((_ECH,), jnp.int32),
        pltpu.VMEM((_ECH,), jnp.int32),
        pltpu.VMEM((_NP,), jnp.float32),
        pltpu.VMEM((_NP,), jnp.float32),
        pltpu.VMEM((_NP,), jnp.float32),
        pltpu.VMEM((_NP,), jnp.float32),
        pltpu.VMEM((_NPT,), jnp.int32),
        pltpu.VMEM((_NPT,), jnp.int32),
        pltpu.VMEM((_NPT,), jnp.int32),
        pltpu.VMEM((_NPT,), jnp.int32),
        pltpu.VMEM((_NPT,), jnp.int32),
        pltpu.VMEM((_NPT,), jnp.float32),
        pltpu.VMEM((_NPT,), jnp.float32),
        pltpu.VMEM((_NPT,), jnp.float32),
        pltpu.VMEM((_NPT,), jnp.float32),
        pltpu.VMEM((_GCH, _D), jnp.float32),
        pltpu.VMEM_SHARED((_NT, _NP), jnp.int32),
        pltpu.VMEM_SHARED((_NT, _NP), jnp.int32),
        pltpu.SemaphoreType.DMA,
    ],
    compiler_params=pltpu.CompilerParams(needs_layout_passes=False),
)


_ACOS_C = (1.5708171339726167, -0.21586769617651358, 0.10738235609240016,
           -0.15901533650606176, 0.34729263730756776, -0.496028713559475,
           0.368623360143272, -0.10906699736254771)
_RAD2DEG = 57.29577951308232


def _acos_deg(x):
  a = jnp.abs(x)
  p = jnp.float32(_ACOS_C[7])
  for k in (6, 5, 4, 3, 2, 1, 0):
    p = p * a + jnp.float32(_ACOS_C[k])
  r = jnp.sqrt(jnp.maximum(1.0 - a, 0.0)) * p
  r = jnp.where(x >= 0, r, jnp.float32(3.141592653589793) - r)
  return r * jnp.float32(_RAD2DEG)


def _tc_body(feat_ref, xg1_ref, xg2_ref, bif_ref, dot_ref, n12_ref, mv_ref,
             wc1a_ref, wc1b_ref, wc1c_ref, bc1_ref, wc2_ref, bc2_ref,
             wr1a_ref, wr1b_ref, wr1c_ref, br1_ref, wr2_ref, br2_ref,
             wr3_ref, br3_ref,
             upd_ref, as_ref, av_ref, mvo_ref, cp_ref):
  x = feat_ref[...]
  xg1 = xg1_ref[...]
  xg2 = xg2_ref[...]
  bif = bif_ref[...]
  dot = dot_ref[...]
  n12 = n12_ref[...]
  mv = mv_ref[...]

  cos = dot / (jnp.sqrt(n12) + 1e-12)
  cos = jnp.clip(cos, -1.0, 1.0)
  ang = _acos_deg(cos)
  in_rng = (ang >= 30.0) & (ang <= 60.0)
  dist = jnp.where(ang < 30.0, 30.0 - ang, ang - 60.0)
  angle_score = jnp.where(in_rng, 1.0, jnp.maximum(0.0, 1.0 - dist / 30.0))
  angle_viol = jnp.where(in_rng, 0.0, 1.0 - angle_score)

  hc = jnp.dot(x, wc1a_ref[...], preferred_element_type=jnp.float32)
  hc += jnp.dot(xg1, wc1b_ref[...], preferred_element_type=jnp.float32)
  hc += jnp.dot(xg2, wc1c_ref[...], preferred_element_type=jnp.float32)
  hc = jnp.maximum(hc + bc1_ref[...], 0.0)
  compliance = jax.nn.sigmoid(
      jnp.sum(hc * wc2_ref[...], axis=1, keepdims=True) + bc2_ref[...])

  hr = jnp.dot(x, wr1a_ref[...], preferred_element_type=jnp.float32)
  hr += jnp.dot(xg1, wr1b_ref[...], preferred_element_type=jnp.float32)
  hr += jnp.dot(xg2, wr1c_ref[...], preferred_element_type=jnp.float32)
  hr = jnp.maximum(hr + br1_ref[...], 0.0)
  hr2 = jnp.maximum(
      jnp.dot(hr, wr2_ref[...], preferred_element_type=jnp.float32)
      + br2_ref[...], 0.0)
  cf = jnp.sum(hr2 * wr3_ref[...], axis=1, keepdims=True) + br3_ref[...]

  needs = (bif > 0.0) & (mv > 0.2)
  corr = cf * jnp.tanh(x)
  upd_ref[...] = x + jnp.where(needs, corr, 0.0)
  as_ref[...] = angle_score * bif
  av_ref[...] = angle_viol * bif
  mvo_ref[...] = mv * bif
  cp_ref[...] = compliance * bif


def _tc_call(featp, xg1, xg2, bifr, dotr, n12r, mvr,
             wc1a, wc1b, wc1c, bc1, wc2, bc2,
             wr1a, wr1b, wr1c, br1, wr2, br2, wr3, br3):
  row = lambda i: (i, 0)
  full = lambda i: (0, 0)
  col = pl.BlockSpec((_BN, 1), row)
  return pl.pallas_call(
      _tc_body,
      grid=(_NB,),
      in_specs=[
          pl.BlockSpec((_BN, _D), row),
          pl.BlockSpec((_BN, _D), row),
          pl.BlockSpec((_BN, _D), row),
          col,
          col,
          col,
          col,
          pl.BlockSpec((_D, 64), full),
          pl.BlockSpec((_D, 64), full),
          pl.BlockSpec((_D, 64), full),
          pl.BlockSpec((1, 64), full),
          pl.BlockSpec((1, 64), full),
          pl.BlockSpec((1, 1), full),
          pl.BlockSpec((_D, 64), full),
          pl.BlockSpec((_D, 64), full),
          pl.BlockSpec((_D, 64), full),
          pl.BlockSpec((1, 64), full),
          pl.BlockSpec((64, 32), full),
          pl.BlockSpec((1, 32), full),
          pl.BlockSpec((1, 32), full),
          pl.BlockSpec((1, 1), full),
      ],
      out_specs=[
          pl.BlockSpec((_BN, _D), row),
          col,
          col,
          col,
          col,
      ],
      out_shape=[
          jax.ShapeDtypeStruct((_NP, _D), jnp.float32),
          jax.ShapeDtypeStruct((_NP, 1), jnp.float32),
          jax.ShapeDtypeStruct((_NP, 1), jnp.float32),
          jax.ShapeDtypeStruct((_NP, 1), jnp.float32),
          jax.ShapeDtypeStruct((_NP, 1), jnp.float32),
      ],
  )(featp, xg1, xg2, bifr, dotr, n12r, mvr,
    wc1a, wc1b, wc1c, bc1, wc2, bc2,
    wr1a, wr1b, wr1c, br1, wr2, br2, wr3, br3)


def kernel(node_features, edge_index, node_positions, node_radii, node_types,
           Wc1, bc1, Wc2, bc2, Wr1, br1, Wr2, br2, Wr3, br3):
  pad = _NP - _N
  src = edge_index[0]
  dst = edge_index[1]
  featp = jnp.pad(node_features, ((0, pad), (0, 0)))
  px = jnp.pad(node_positions[:, 0], (0, pad))
  py = jnp.pad(node_positions[:, 1], (0, pad))
  pz = jnp.pad(node_positions[:, 2], (0, pad))
  rr = jnp.pad(node_radii, (0, pad))
  ty = jnp.pad(node_types, (0, pad))

  bif, dotv, n12, mv, xg1, xg2 = _sc_call(src, dst, px, py, pz, rr, ty, featp)

  shp = (_NP, 1)
  upd, asg, avg, mvg, cpg = _tc_call(
      featp, xg1, xg2,
      bif.reshape(shp), dotv.reshape(shp), n12.reshape(shp), mv.reshape(shp),
      Wc1[:_D], Wc1[_D:2 * _D], Wc1[2 * _D:], bc1.reshape(1, 64),
      Wc2.reshape(1, 64), bc2.reshape(1, 1),
      Wr1[:_D], Wr1[_D:2 * _D], Wr1[2 * _D:], br1.reshape(1, 64),
      Wr2, br2.reshape(1, 32), Wr3.reshape(1, 32), br3.reshape(1, 1))

  return (upd[:_N],
          asg.reshape(_NP)[:_N],
          avg.reshape(_NP)[:_N],
          mvg.reshape(_NP)[:_N],
          cpg.reshape(_NP)[:_N])


# speedup vs baseline: 2.8657x; 1.0424x over previous
import functools

import jax
import jax.numpy as jnp
from jax import lax
from jax.experimental import pallas as pl
from jax.experimental.pallas import tpu as pltpu
from jax.experimental.pallas import tpu_sc as plsc

_N = 10000
_E = 320000
_D = 128
_NP = 10240
_NT = 16
_EPT = _E // _NT
_ECH = 4000
_NPT = _NP // _NT
_GCH = 128

_BN = 1000
_NB = _N // _BN

_SC_MESH = plsc.VectorSubcoreMesh(
    core_axis_name="c", subcore_axis_name="s", num_cores=1, num_subcores=_NT)


def _sc_body(src_hbm, dst_hbm, px_hbm, py_hbm, pz_hbm, rr_hbm, ty_hbm,
             feat_hbm,
             bif_hbm, dot_hbm, n12_hbm, mv_hbm, xg1_hbm, xg2_hbm,
             c1_loc, c2_loc, srcb, dstb, pxt, pyt, pzt, rrt,
             tyb, pb1, pb2, mc1, mc2, bifb, dotb, n12b, mvb, rows, rows2,
             sh1, sh2, sem, gsem, tsem):
  tid = lax.axis_index("s")
  nb = tid * _NPT
  neg1 = jnp.full((16,), -1, jnp.int32)

  tcp1 = pltpu.async_copy(px_hbm, pxt, tsem)
  tcp2 = pltpu.async_copy(py_hbm, pyt, tsem)
  tcp3 = pltpu.async_copy(pz_hbm, pzt, tsem)
  tcp4 = pltpu.async_copy(rr_hbm, rrt, tsem)
  tcp5 = pltpu.async_copy(ty_hbm.at[pl.ds(nb, _NPT)], tyb, tsem)

  def init_loop(i, _):
    c1_loc[pl.ds(i * 16, 16)] = neg1
    c2_loc[pl.ds(i * 16, 16)] = neg1
    return 0

  lax.fori_loop(0, _NP // 16, init_loop, 0)

  ebase = tid * _EPT

  def chunk_loop(ci, _):
    off = ebase + ci * _ECH
    pltpu.sync_copy(src_hbm.at[pl.ds(off, _ECH)], srcb)
    pltpu.sync_copy(dst_hbm.at[pl.ds(off, _ECH)], dstb)

    def vec_loop(vi, _):
      sl = pl.ds(vi * 16, 16)
      s = srcb[sl]
      d = dstb[sl]
      occ, _last = plsc.scan_count(s)
      cur1 = plsc.load_gather(c1_loc, [s])
      cur2 = plsc.load_gather(c2_loc, [s])
      e1 = cur1 < 0
      e2 = cur2 < 0
      m1 = e1 & (occ == 1)
      m2 = ((~e1) & e2 & (occ == 1)) | (e1 & (occ == 2))
      plsc.store_scatter(c1_loc, [s], d, mask=m1)
      plsc.store_scatter(c2_loc, [s], d, mask=m2)
      return 0

    lax.fori_loop(0, _ECH // 16, vec_loop, 0)
    return 0

  lax.fori_loop(0, _EPT // _ECH, chunk_loop, 0)

  pltpu.sync_copy(c1_loc, sh1.at[tid])
  pltpu.sync_copy(c2_loc, sh2.at[tid])
  plsc.subcore_barrier()


  def minit(i, _):
    mc1[pl.ds(i * 16, 16)] = neg1
    mc2[pl.ds(i * 16, 16)] = neg1
    return 0

  lax.fori_loop(0, _NPT // 16, minit, 0)

  def tmerge(tp, _):
    pltpu.sync_copy(sh1.at[tp, pl.ds(nb, _NPT)], pb1)
    pltpu.sync_copy(sh2.at[tp, pl.ds(nb, _NPT)], pb2)

    def mvec(vi, _):
      sl = pl.ds(vi * 16, 16)
      a = pb1[sl]
      b = pb2[sl]
      c1v = mc1[sl]
      c2v = mc2[sl]
      mc1[sl] = jnp.where(c1v < 0, a, c1v)
      mc2[sl] = jnp.where(c1v < 0, b, jnp.where(c2v < 0, a, c2v))
      return 0

    lax.fori_loop(0, _NPT // 16, mvec, 0)
    return 0

  lax.fori_loop(0, _NT, tmerge, 0)

  tcp1.wait()
  tcp2.wait()
  tcp3.wait()
  tcp4.wait()
  tcp5.wait()

  def fvec(vi, _):
    sl = pl.ds(vi * 16, 16)
    osl = pl.ds(nb + vi * 16, 16)
    c1v = mc1[sl]
    c2v = mc2[sl]
    tyv = tyb[sl]
    c1f = jnp.maximum(c1v, 0)
    c2f = jnp.where(c2v < 0, c1f, c2v)
    mc1[sl] = c1f
    mc2[sl] = c2f
    bifb[sl] = jnp.where((tyv == 1) & (c2v >= 0), 1.0, 0.0)
    p1x = plsc.load_gather(pxt, [c1f])
    p2x = plsc.load_gather(pxt, [c2f])
    p1y = plsc.load_gather(pyt, [c1f])
    p2y = plsc.load_gather(pyt, [c2f])
    p1z = plsc.load_gather(pzt, [c1f])
    p2z = plsc.load_gather(pzt, [c2f])
    ox = pxt[osl]
    oy = pyt[osl]
    oz = pzt[osl]
    v1x = p1x - ox
    v1y = p1y - oy
    v1z = p1z - oz
    v2x = p2x - ox
    v2y = p2y - oy
    v2z = p2z - oz
    dotb[sl] = v1x * v2x + v1y * v2y + v1z * v2z
    n12b[sl] = ((v1x * v1x + v1y * v1y + v1z * v1z) *
                (v2x * v2x + v2y * v2y + v2z * v2z))
    r1 = plsc.load_gather(rrt, [c1f])
    r2 = plsc.load_gather(rrt, [c2f])
    ro = rrt[osl]
    act = ro * ro * ro
    mvb[sl] = jnp.abs(r1 * r1 * r1 + r2 * r2 * r2 - act) / (act + 1e-12)
    return 0

  lax.fori_loop(0, _NPT // 16, fvec, 0)

  pltpu.sync_copy(bifb, bif_hbm.at[pl.ds(nb, _NPT)])
  pltpu.sync_copy(dotb, dot_hbm.at[pl.ds(nb, _NPT)])
  pltpu.sync_copy(n12b, n12_hbm.at[pl.ds(nb, _NPT)])
  pltpu.sync_copy(mvb, mv_hbm.at[pl.ds(nb, _NPT)])

  nch = _NPT // _GCH
  plans = []
  for ch in range(nch):
    plans.append((mc1, xg1_hbm, ch))
    plans.append((mc2, xg2_hbm, ch))
  bufs = (rows, rows2)
  cps = [None, None]
  for i, (idx_ref, out_hbm, ch) in enumerate(plans):
    b = bufs[i % 2]
    if cps[i % 2] is not None:
      cps[i % 2].wait()
    cps[i % 2] = None
    pltpu.async_copy(
        feat_hbm.at[idx_ref.at[pl.ds(ch * _GCH, _GCH)]], b, gsem).wait()
    cps[i % 2] = pltpu.async_copy(
        b, out_hbm.at[pl.ds(nb + ch * _GCH, _GCH)], sem)
  for c in cps:
    if c is not None:
      c.wait()


_sc_call = pl.kernel(
    _sc_body,
    out_type=[
        jax.ShapeDtypeStruct((_NP,), jnp.float32),
        jax.ShapeDtypeStruct((_NP,), jnp.float32),
        jax.ShapeDtypeStruct((_NP,), jnp.float32),
        jax.ShapeDtypeStruct((_NP,), jnp.float32),
        jax.ShapeDtypeStruct((_NP, _D), jnp.float32),
        jax.ShapeDtypeStruct((_NP, _D), jnp.float32),
    ],
    mesh=_SC_MESH,
    scratch_types=[
        pltpu.VMEM((_NP,), jnp.int32),
        pltpu.VMEM((_NP,), jnp.int32),
        pltpu.VMEM((_ECH,), jnp.int32),
        pltpu.VMEM((_ECH,), jnp.int32),
        pltpu.VMEM((_NP,), jnp.float32),
        pltpu.VMEM((_NP,), jnp.float32),
        pltpu.VMEM((_NP,), jnp.float32),
        pltpu.VMEM((_NP,), jnp.float32),
        pltpu.VMEM((_NPT,), jnp.int32),
        pltpu.VMEM((_NPT,), jnp.int32),
        pltpu.VMEM((_NPT,), jnp.int32),
        pltpu.VMEM((_NPT,), jnp.int32),
        pltpu.VMEM((_NPT,), jnp.int32),
        pltpu.VMEM((_NPT,), jnp.float32),
        pltpu.VMEM((_NPT,), jnp.float32),
        pltpu.VMEM((_NPT,), jnp.float32),
        pltpu.VMEM((_NPT,), jnp.float32),
        pltpu.VMEM((_GCH, _D), jnp.float32),
        pltpu.VMEM((_GCH, _D), jnp.float32),
        pltpu.VMEM_SHARED((_NT, _NP), jnp.int32),
        pltpu.VMEM_SHARED((_NT, _NP), jnp.int32),
        pltpu.SemaphoreType.DMA,
        pltpu.SemaphoreType.DMA,
        pltpu.SemaphoreType.DMA,
    ],
    compiler_params=pltpu.CompilerParams(needs_layout_passes=False),
)


_ACOS_C = (1.5708171339726167, -0.21586769617651358, 0.10738235609240016,
           -0.15901533650606176, 0.34729263730756776, -0.496028713559475,
           0.368623360143272, -0.10906699736254771)
_RAD2DEG = 57.29577951308232


def _acos_deg(x):
  a = jnp.abs(x)
  p = jnp.float32(_ACOS_C[7])
  for k in (6, 5, 4, 3, 2, 1, 0):
    p = p * a + jnp.float32(_ACOS_C[k])
  r = jnp.sqrt(jnp.maximum(1.0 - a, 0.0)) * p
  r = jnp.where(x >= 0, r, jnp.float32(3.141592653589793) - r)
  return r * jnp.float32(_RAD2DEG)


def _tc_body(feat_ref, xg1_ref, xg2_ref, bif_ref, dot_ref, n12_ref, mv_ref,
             wc1a_ref, wc1b_ref, wc1c_ref, bc1_ref, wc2_ref, bc2_ref,
             wr1a_ref, wr1b_ref, wr1c_ref, br1_ref, wr2_ref, br2_ref,
             wr3_ref, br3_ref,
             upd_ref, as_ref, av_ref, mvo_ref, cp_ref):
  x = feat_ref[...]
  xg1 = xg1_ref[...]
  xg2 = xg2_ref[...]
  bif = bif_ref[...]
  dot = dot_ref[...]
  n12 = n12_ref[...]
  mv = mv_ref[...]

  cos = dot / (jnp.sqrt(n12) + 1e-12)
  cos = jnp.clip(cos, -1.0, 1.0)
  ang = _acos_deg(cos)
  in_rng = (ang >= 30.0) & (ang <= 60.0)
  dist = jnp.where(ang < 30.0, 30.0 - ang, ang - 60.0)
  angle_score = jnp.where(in_rng, 1.0, jnp.maximum(0.0, 1.0 - dist / 30.0))
  angle_viol = jnp.where(in_rng, 0.0, 1.0 - angle_score)

  hc = jnp.dot(x, wc1a_ref[...], preferred_element_type=jnp.float32)
  hc += jnp.dot(xg1, wc1b_ref[...], preferred_element_type=jnp.float32)
  hc += jnp.dot(xg2, wc1c_ref[...], preferred_element_type=jnp.float32)
  hc = jnp.maximum(hc + bc1_ref[...], 0.0)
  compliance = jax.nn.sigmoid(
      jnp.sum(hc * wc2_ref[...], axis=1, keepdims=True) + bc2_ref[...])

  hr = jnp.dot(x, wr1a_ref[...], preferred_element_type=jnp.float32)
  hr += jnp.dot(xg1, wr1b_ref[...], preferred_element_type=jnp.float32)
  hr += jnp.dot(xg2, wr1c_ref[...], preferred_element_type=jnp.float32)
  hr = jnp.maximum(hr + br1_ref[...], 0.0)
  hr2 = jnp.maximum(
      jnp.dot(hr, wr2_ref[...], preferred_element_type=jnp.float32)
      + br2_ref[...], 0.0)
  cf = jnp.sum(hr2 * wr3_ref[...], axis=1, keepdims=True) + br3_ref[...]

  needs = (bif > 0.0) & (mv > 0.2)
  corr = cf * jnp.tanh(x)
  upd_ref[...] = x + jnp.where(needs, corr, 0.0)
  as_ref[...] = angle_score * bif
  av_ref[...] = angle_viol * bif
  mvo_ref[...] = mv * bif
  cp_ref[...] = compliance * bif


def _tc_call(featp, xg1, xg2, bifr, dotr, n12r, mvr,
             wc1a, wc1b, wc1c, bc1, wc2, bc2,
             wr1a, wr1b, wr1c, br1, wr2, br2, wr3, br3):
  row = lambda i: (i, 0)
  full = lambda i: (0, 0)
  col = pl.BlockSpec((_BN, 1), row)
  return pl.pallas_call(
      _tc_body,
      grid=(_NB,),
      in_specs=[
          pl.BlockSpec((_BN, _D), row),
          pl.BlockSpec((_BN, _D), row),
          pl.BlockSpec((_BN, _D), row),
          col,
          col,
          col,
          col,
          pl.BlockSpec((_D, 64), full),
          pl.BlockSpec((_D, 64), full),
          pl.BlockSpec((_D, 64), full),
          pl.BlockSpec((1, 64), full),
          pl.BlockSpec((1, 64), full),
          pl.BlockSpec((1, 1), full),
          pl.BlockSpec((_D, 64), full),
          pl.BlockSpec((_D, 64), full),
          pl.BlockSpec((_D, 64), full),
          pl.BlockSpec((1, 64), full),
          pl.BlockSpec((64, 32), full),
          pl.BlockSpec((1, 32), full),
          pl.BlockSpec((1, 32), full),
          pl.BlockSpec((1, 1), full),
      ],
      out_specs=[
          pl.BlockSpec((_BN, _D), row),
          col,
          col,
          col,
          col,
      ],
      out_shape=[
          jax.ShapeDtypeStruct((_N, _D), jnp.float32),
          jax.ShapeDtypeStruct((_N, 1), jnp.float32),
          jax.ShapeDtypeStruct((_N, 1), jnp.float32),
          jax.ShapeDtypeStruct((_N, 1), jnp.float32),
          jax.ShapeDtypeStruct((_N, 1), jnp.float32),
      ],
  )(featp, xg1, xg2, bifr, dotr, n12r, mvr,
    wc1a, wc1b, wc1c, bc1, wc2, bc2,
    wr1a, wr1b, wr1c, br1, wr2, br2, wr3, br3)


def kernel(node_features, edge_index, node_positions, node_radii, node_types,
           Wc1, bc1, Wc2, bc2, Wr1, br1, Wr2, br2, Wr3, br3):
  pad = _NP - _N
  src = edge_index[0]
  dst = edge_index[1]
  px = jnp.pad(node_positions[:, 0], (0, pad))
  py = jnp.pad(node_positions[:, 1], (0, pad))
  pz = jnp.pad(node_positions[:, 2], (0, pad))
  rr = jnp.pad(node_radii, (0, pad))
  ty = jnp.pad(node_types, (0, pad))

  bif, dotv, n12, mv, xg1, xg2 = _sc_call(src, dst, px, py, pz, rr, ty,
                                          node_features)

  shp = (_NP, 1)
  upd, asg, avg, mvg, cpg = _tc_call(
      node_features, xg1, xg2,
      bif.reshape(shp), dotv.reshape(shp), n12.reshape(shp), mv.reshape(shp),
      Wc1[:_D], Wc1[_D:2 * _D], Wc1[2 * _D:], bc1.reshape(1, 64),
      Wc2.reshape(1, 64), bc2.reshape(1, 1),
      Wr1[:_D], Wr1[_D:2 * _D], Wr1[2 * _D:], br1.reshape(1, 64),
      Wr2, br2.reshape(1, 32), Wr3.reshape(1, 32), br3.reshape(1, 1))

  return (upd, asg.reshape(_N), avg.reshape(_N), mvg.reshape(_N),
          cpg.reshape(_N))
